# baseline (speedup 1.0000x reference)
import functools

import jax
import jax.numpy as jnp
from jax import lax
from jax.experimental import pallas as pl
from jax.experimental.pallas import tpu as pltpu
from jax.experimental.pallas import tpu_sc as plsc

N_USERS = 25000
N_ITEMS = 25000
N = N_USERS + N_ITEMS
NPAD = 50048
D = 64
Q = 16
NQ = D // Q
E = 800000
NC = 2
NS = 16
EDGES_PER_TILE = E // NS
CHUNK = 1000
NCHUNKS = EDGES_PER_TILE // CHUNK
ROWS_PER_TILE = NPAD // NS
WB_SIZES = (632, 632, 632, 632, 600)
L = 16
TBL = N * NQ
TBL_VIEW = TBL - (NQ - 1)


def _scale_rows(rows_ref, w_ref):
    @plsc.parallel_loop(0, CHUNK // L, unroll=2)
    def _(g):
        e0 = g * L
        w16 = w_ref[pl.ds(e0, L)]
        for k in range(L):
            rows_ref[e0 + k, pl.ds(0, L)] = rows_ref[e0 + k, pl.ds(0, L)] * w16[k]
    rem = CHUNK % L
    if rem:
        w16 = w_ref[pl.ds(CHUNK - L, L)]
        for k in range(L - rem, L):
            e = CHUNK - L + k
            rows_ref[e, pl.ds(0, L)] = rows_ref[e, pl.ds(0, L)] * w16[k]


def _third_rows(a_ref, n):
    third = jnp.float32(1.0 / 3.0)

    @plsc.parallel_loop(0, n, unroll=4)
    def _(r):
        a_ref[r, pl.ds(0, L)] = a_ref[r, pl.ds(0, L)] * third


def _make_kernel():
    mesh = plsc.VectorSubcoreMesh(core_axis_name="c", subcore_axis_name="s")

    scratch = (
        [pltpu.VMEM((CHUNK, Q), jnp.float32)] * 2
        + [pltpu.VMEM((2, CHUNK), jnp.int32)] * 4
        + [pltpu.VMEM((CHUNK,), jnp.float32)] * 4
        + [pltpu.VMEM((WB_SIZES[0], Q), jnp.float32)]
        + [pltpu.VMEM_SHARED((NPAD, Q), jnp.float32)]
        + [pltpu.SemaphoreType.DMA] * 8
    )

    @functools.partial(
        pl.kernel,
        mesh=mesh,
        out_type=[
            jax.ShapeDtypeStruct((NQ, NPAD, Q), jnp.float32),
            jax.ShapeDtypeStruct((NQ, NPAD, Q), jnp.float32),
        ],
        scratch_types=scratch,
        compiler_params=pltpu.CompilerParams(use_tc_tiling_on_sc=False),
    )
    def kern(ego0f, pk1, pk2, ws, ego1, out, *sc):
        rows = sc[0:2]
        sd = sc[2:6]
        wv = sc[6:10]
        wb = sc[10]
        acc = sc[11]
        sem_g = sc[12:14]
        sem_i = sc[14:18]
        sem_s = sc[18:20]

        c = lax.axis_index("c")
        s = lax.axis_index("s")
        ebase = s * EDGES_PER_TILE
        rbase = s * ROWS_PER_TILE

        def issue_idx(j, base, layer1):
            pk = pk1 if layer1 else pk2
            pltpu.async_copy(pk.at[:, pl.ds(base, CHUNK)], sd[j], sem_i[j])
            pltpu.async_copy(ws.at[pl.ds(base, CHUNK)], wv[j], sem_i[j])

        def wait_idx(j):
            pltpu.make_async_copy(pk2.at[:, pl.ds(0, CHUNK)], sd[j], sem_i[j]).wait()
            pltpu.make_async_copy(ws.at[pl.ds(0, CHUNK)], wv[j], sem_i[j]).wait()

        def wait_rowsized(sem, rb):
            pltpu.make_async_copy(ego0f.at[pl.ds(0, CHUNK)], rows[rb], sem).wait()

        def layer(tab, layer1, q):
            def issue_gather(j, rb):
                pltpu.async_copy(tab.at[sd[j].at[0]], rows[rb], sem_g[rb])

            def issue_scatter(jb, rb):
                pltpu.async_copy(rows[rb], acc.at[sd[jb].at[1]], sem_s[rb], add=True)

            issue_idx(0, ebase, layer1)
            issue_idx(1, ebase + CHUNK, layer1)
            wait_idx(0)
            issue_gather(0, 0)
            wait_rowsized(sem_g[0], 0)
            wait_idx(1)
            issue_gather(1, 1)
            issue_idx(2, ebase + 2 * CHUNK, layer1)
            _scale_rows(rows[0], wv[0])
            issue_scatter(0, 0)

            def g_body(g, _):
                for k in range(4):
                    i = 1 + g * 4 + k
                    rb = (1 + k) % 2
                    jb = (1 + k) % 4
                    jn1 = (2 + k) % 4
                    jn2 = (3 + k) % 4
                    wait_rowsized(sem_g[rb], rb)
                    wait_rowsized(sem_s[1 - rb], 1 - rb)

                    @pl.when(i + 1 < NCHUNKS)
                    def _():
                        wait_idx(jn1)
                        issue_gather(jn1, 1 - rb)

                    @pl.when(i + 2 < NCHUNKS)
                    def _():
                        issue_idx(jn2, ebase + (i + 2) * CHUNK, layer1)

                    _scale_rows(rows[rb], wv[jb])
                    issue_scatter(jb, rb)
                return 0

            lax.fori_loop(0, (NCHUNKS - 2) // 4, g_body, 0)
            wait_rowsized(sem_g[1], 1)
            wait_rowsized(sem_s[0], 0)
            _scale_rows(rows[1], wv[1])
            issue_scatter(1, 1)
            wait_rowsized(sem_s[1], 1)

        for p in range(2):
            q = 2 * p + c

            @plsc.parallel_loop(0, CHUNK, unroll=4)
            def _(i):
                rows[0][i, pl.ds(0, L)] = jnp.zeros((L,), jnp.float32)
            nfull = ROWS_PER_TILE // CHUNK
            for j in range(nfull):
                pltpu.sync_copy(rows[0], acc.at[pl.ds(rbase + j * CHUNK, CHUNK)])
            ztail = ROWS_PER_TILE - nfull * CHUNK
            if ztail:
                pltpu.sync_copy(rows[0].at[pl.ds(0, ztail)],
                                acc.at[pl.ds(rbase + nfull * CHUNK, ztail)])
            plsc.subcore_barrier()

            layer(ego0f.at[pl.ds(q, TBL_VIEW)], True, q)
            plsc.subcore_barrier()

            for j in range(nfull):
                sl = pl.ds(rbase + j * CHUNK, CHUNK)
                pltpu.sync_copy(acc.at[sl], ego1.at[q, sl])
            if ztail:
                tail = pl.ds(rbase + nfull * CHUNK, ztail)
                pltpu.sync_copy(acc.at[tail], ego1.at[q, tail])
            plsc.subcore_barrier()

            layer(ego1.at[q], False, q)
            plsc.subcore_barrier()

            off = 0
            for nwb in WB_SIZES:
                sl = pl.ds(rbase + off, nwb)
                vsl = pl.ds(0, nwb)
                pltpu.sync_copy(acc.at[sl], wb.at[vsl])
                _third_rows(wb, nwb)
                pltpu.sync_copy(wb.at[vsl], out.at[q, sl])
                off += nwb

    return kern


_kernel = _make_kernel()


@jax.jit
def kernel(user_emb, item_emb, edge_weight, edge_index):
    ego = jnp.concatenate([user_emb, item_emb], axis=0)
    ego0f = ego.reshape(TBL, Q)
    src = edge_index[0]
    dst = edge_index[1]
    pk1 = jnp.stack([src * 4, dst], axis=0)
    pk2 = jnp.stack([src, dst], axis=0)
    _, out = _kernel(ego0f, pk1, pk2, edge_weight)
    prop = jnp.moveaxis(out, 0, 1).reshape(NPAD, D)[:N]
    final = ego * jnp.float32(1.0 / 3.0) + prop
    return (final[:N_USERS], final[N_USERS:])


# speedup vs baseline: 1.0028x; 1.0028x over previous
import functools

import jax
import jax.numpy as jnp
from jax import lax
from jax.experimental import pallas as pl
from jax.experimental.pallas import tpu as pltpu
from jax.experimental.pallas import tpu_sc as plsc

N_USERS = 25000
N_ITEMS = 25000
N = N_USERS + N_ITEMS
NPAD = 50048
D = 64
Q = 16
NQ = D // Q
E = 800000
NC = 2
NS = 16
EDGES_PER_TILE = E // NS
CHUNK = 1000
NCHUNKS = EDGES_PER_TILE // CHUNK
ROWS_PER_TILE = NPAD // NS
WB_SIZES = (632, 632, 632, 632, 600)
L = 16
TBL = N * NQ
TBL_VIEW = TBL - (NQ - 1)


def _scale_rows(rows_ref, w_ref):
    @plsc.parallel_loop(0, CHUNK // L, unroll=2)
    def _(g):
        e0 = g * L
        w16 = w_ref[pl.ds(e0, L)]
        for k in range(L):
            rows_ref[e0 + k, pl.ds(0, L)] = rows_ref[e0 + k, pl.ds(0, L)] * w16[k]
    rem = CHUNK % L
    if rem:
        w16 = w_ref[pl.ds(CHUNK - L, L)]
        for k in range(L - rem, L):
            e = CHUNK - L + k
            rows_ref[e, pl.ds(0, L)] = rows_ref[e, pl.ds(0, L)] * w16[k]


def _third_rows(a_ref, n):
    third = jnp.float32(1.0 / 3.0)

    @plsc.parallel_loop(0, n, unroll=4)
    def _(r):
        a_ref[r, pl.ds(0, L)] = a_ref[r, pl.ds(0, L)] * third


def _make_kernel():
    mesh = plsc.VectorSubcoreMesh(core_axis_name="c", subcore_axis_name="s")

    scratch = (
        [pltpu.VMEM((CHUNK, Q), jnp.float32)] * 2
        + [pltpu.VMEM((2, CHUNK), jnp.int32)] * 4
        + [pltpu.VMEM((CHUNK,), jnp.float32)] * 4
        + [pltpu.VMEM((WB_SIZES[0], Q), jnp.float32)]
        + [pltpu.VMEM_SHARED((NPAD, Q), jnp.float32)]
        + [pltpu.SemaphoreType.DMA] * 8
    )

    @functools.partial(
        pl.kernel,
        mesh=mesh,
        out_type=[
            jax.ShapeDtypeStruct((NQ, NPAD, Q), jnp.float32),
            jax.ShapeDtypeStruct((NQ, NPAD, Q), jnp.float32),
        ],
        scratch_types=scratch,
        compiler_params=pltpu.CompilerParams(use_tc_tiling_on_sc=False),
    )
    def kern(ego0f, pk1, pk2, ws, ego1, out, *sc):
        rows = sc[0:2]
        sd = sc[2:6]
        wv = sc[6:10]
        wb = sc[10]
        acc = sc[11]
        sem_g = sc[12:14]
        sem_i = sc[14:18]
        sem_s = sc[18:20]

        c = lax.axis_index("c")
        s = lax.axis_index("s")
        ebase = s * EDGES_PER_TILE
        rbase = s * ROWS_PER_TILE

        def issue_idx(j, base, layer1):
            pk = pk1 if layer1 else pk2
            pltpu.async_copy(pk.at[:, pl.ds(base, CHUNK)], sd[j], sem_i[j])
            pltpu.async_copy(ws.at[pl.ds(base, CHUNK)], wv[j], sem_i[j])

        def wait_idx(j):
            pltpu.make_async_copy(pk2.at[:, pl.ds(0, CHUNK)], sd[j], sem_i[j]).wait()
            pltpu.make_async_copy(ws.at[pl.ds(0, CHUNK)], wv[j], sem_i[j]).wait()

        def wait_rowsized(sem, rb):
            pltpu.make_async_copy(ego0f.at[pl.ds(0, CHUNK)], rows[rb], sem).wait()

        def layer(tab, layer1, q):
            def issue_gather(j, rb):
                pltpu.async_copy(tab.at[sd[j].at[0]], rows[rb], sem_g[rb])

            def issue_scatter(jb, rb):
                pltpu.async_copy(rows[rb], acc.at[sd[jb].at[1]], sem_s[rb], add=True)

            issue_idx(0, ebase, layer1)
            issue_idx(1, ebase + CHUNK, layer1)
            wait_idx(0)
            issue_gather(0, 0)
            wait_rowsized(sem_g[0], 0)
            wait_idx(1)
            issue_gather(1, 1)
            issue_idx(2, ebase + 2 * CHUNK, layer1)
            _scale_rows(rows[0], wv[0])
            issue_scatter(0, 0)

            def g_body(g, _):
                for k in range(4):
                    i = 1 + g * 4 + k
                    rb = (1 + k) % 2
                    jb = (1 + k) % 4
                    jn1 = (2 + k) % 4
                    jn2 = (3 + k) % 4
                    wait_rowsized(sem_g[rb], rb)
                    wait_rowsized(sem_s[1 - rb], 1 - rb)

                    @pl.when(i + 1 < NCHUNKS)
                    def _():
                        wait_idx(jn1)
                        issue_gather(jn1, 1 - rb)

                    @pl.when(i + 2 < NCHUNKS)
                    def _():
                        issue_idx(jn2, ebase + (i + 2) * CHUNK, layer1)

                    _scale_rows(rows[rb], wv[jb])
                    issue_scatter(jb, rb)
                return 0

            lax.fori_loop(0, (NCHUNKS - 2) // 4, g_body, 0)
            wait_rowsized(sem_g[1], 1)
            wait_rowsized(sem_s[0], 0)
            _scale_rows(rows[1], wv[1])
            issue_scatter(1, 1)
            wait_rowsized(sem_s[1], 1)

        def pass_body(p, _):
            q = 2 * p + c

            @plsc.parallel_loop(0, CHUNK, unroll=4)
            def _(i):
                rows[0][i, pl.ds(0, L)] = jnp.zeros((L,), jnp.float32)
            nfull = ROWS_PER_TILE // CHUNK
            for j in range(nfull):
                pltpu.sync_copy(rows[0], acc.at[pl.ds(rbase + j * CHUNK, CHUNK)])
            ztail = ROWS_PER_TILE - nfull * CHUNK
            if ztail:
                pltpu.sync_copy(rows[0].at[pl.ds(0, ztail)],
                                acc.at[pl.ds(rbase + nfull * CHUNK, ztail)])
            plsc.subcore_barrier()

            layer(ego0f.at[pl.ds(q, TBL_VIEW)], True, q)
            plsc.subcore_barrier()

            for j in range(nfull):
                sl = pl.ds(rbase + j * CHUNK, CHUNK)
                pltpu.sync_copy(acc.at[sl], ego1.at[q, sl])
            if ztail:
                tail = pl.ds(rbase + nfull * CHUNK, ztail)
                pltpu.sync_copy(acc.at[tail], ego1.at[q, tail])
            plsc.subcore_barrier()

            layer(ego1.at[q], False, q)
            plsc.subcore_barrier()

            off = 0
            for nwb in WB_SIZES:
                sl = pl.ds(rbase + off, nwb)
                vsl = pl.ds(0, nwb)
                pltpu.sync_copy(acc.at[sl], wb.at[vsl])
                _third_rows(wb, nwb)
                pltpu.sync_copy(wb.at[vsl], out.at[q, sl])
                off += nwb
            return 0

        lax.fori_loop(0, 2, pass_body, 0)

    return kern


_kernel = _make_kernel()


@jax.jit
def kernel(user_emb, item_emb, edge_weight, edge_index):
    ego = jnp.concatenate([user_emb, item_emb], axis=0)
    ego0f = ego.reshape(TBL, Q)
    src = edge_index[0]
    dst = edge_index[1]
    pk1 = jnp.stack([src * 4, dst], axis=0)
    pk2 = jnp.stack([src, dst], axis=0)
    _, out = _kernel(ego0f, pk1, pk2, edge_weight)
    prop = jnp.moveaxis(out, 0, 1).reshape(NPAD, D)[:N]
    final = ego * jnp.float32(1.0 / 3.0) + prop
    return (final[:N_USERS], final[N_USERS:])
